# baseline (speedup 1.0000x reference)
import functools

import jax
import jax.numpy as jnp
from jax import lax
from jax.experimental import pallas as pl
from jax.experimental.pallas import tpu as pltpu
from jax.experimental.pallas import tpu_sc as plsc

NUM_ITEMS = 1000000
LATENT = 64
BATCH = 16384

NC = 2
NS = 16
NW = NC * NS
B_PER_W = BATCH // NW
SUB = 8
N_TILES = NUM_ITEMS // SUB


def _make_gather():
  mesh = plsc.VectorSubcoreMesh(
      core_axis_name="c", subcore_axis_name="s", num_cores=NC,
      num_subcores=NS)

  @functools.partial(
      pl.kernel,
      mesh=mesh,
      compiler_params=pltpu.CompilerParams(use_tc_tiling_on_sc=True),
      out_type=jax.ShapeDtypeStruct((BATCH, LATENT), jnp.float32),
      scratch_types=[
          pltpu.VMEM((B_PER_W,), jnp.int32),
          pltpu.VMEM((B_PER_W, LATENT), jnp.float32),
          pltpu.SemaphoreType.DMA,
      ],
  )
  def gather_k(idx_hbm, table_hbm, out_hbm, idx_v, ext_v, sem):
    wid = lax.axis_index("s") * NC + lax.axis_index("c")
    base = wid * B_PER_W
    pltpu.sync_copy(idx_hbm.at[pl.ds(base, B_PER_W)], idx_v)

    def issue(g, _):
      vec = idx_v[pl.ds(g * 16, 16)]
      for k in range(16):
        ij = vec[k]
        t = ij // SUB
        r = ij % SUB
        pltpu.async_copy(table_hbm.at[t, r], ext_v.at[g * 16 + k], sem)
      return _

    lax.fori_loop(0, B_PER_W // 16, issue, None)
    pltpu.make_async_copy(
        out_hbm.at[pl.ds(base, B_PER_W)], ext_v, sem).wait()
    pltpu.sync_copy(ext_v, out_hbm.at[pl.ds(base, B_PER_W)])

  return gather_k


_gather = _make_gather()

_TC_BLOCK = 1024


def _tc_body(x_ref, w_ref, b_ref, o_ref):
  x = x_ref[...]
  w = w_ref[...]
  s = jnp.sum(x * w, axis=1, keepdims=True) + b_ref[0, 0]
  o_ref[...] = jax.nn.sigmoid(s)


def _dense_stage(rows, W, b):
  grid = (BATCH // _TC_BLOCK,)
  return pl.pallas_call(
      _tc_body,
      grid=grid,
      in_specs=[
          pl.BlockSpec((_TC_BLOCK, LATENT), lambda i: (i, 0)),
          pl.BlockSpec((1, LATENT), lambda i: (0, 0)),
          pl.BlockSpec((1, 1), lambda i: (0, 0)),
      ],
      out_specs=pl.BlockSpec((_TC_BLOCK, 1), lambda i: (i, 0)),
      out_shape=jax.ShapeDtypeStruct((BATCH, 1), jnp.float32),
  )(rows, W, b.reshape(1, 1))


@jax.jit
def kernel(item_indices, emb_table, W, b):
  idx0 = (item_indices - 1).astype(jnp.int32)
  table3 = emb_table.reshape(N_TILES, SUB, LATENT)
  rows = _gather(idx0, table3)
  out = _dense_stage(rows, W, b)
  return out.reshape(BATCH)


# speedup vs baseline: 1.2074x; 1.2074x over previous
import functools

import jax
import jax.numpy as jnp
from jax import lax
from jax.experimental import pallas as pl
from jax.experimental.pallas import tpu as pltpu
from jax.experimental.pallas import tpu_sc as plsc

NUM_ITEMS = 1000000
LATENT = 64
BATCH = 16384

NC = 2
NS = 16
NW = NC * NS
B_PER_W = BATCH // NW

SC_COLS = 262144
SLAB = 128
NSLAB = SC_COLS // (SLAB * NW)
TC_COLS = NUM_ITEMS - SC_COLS

_BLK = 32768


def _mesh():
  return plsc.VectorSubcoreMesh(
      core_axis_name="c", subcore_axis_name="s", num_cores=NC,
      num_subcores=NS)


def _mv_body(xT_ref, w_ref, o_ref):
  x = xT_ref[...]
  w = w_ref[...]
  o_ref[...] = jnp.sum(x * w, axis=0)


def _tc_sweep(tableT, W):
  grid = (pl.cdiv(TC_COLS, _BLK),)
  return pl.pallas_call(
      _mv_body,
      grid=grid,
      in_specs=[
          pl.BlockSpec((LATENT, _BLK), lambda i: (0, i + SC_COLS // _BLK)),
          pl.BlockSpec((LATENT, 1), lambda i: (0, 0)),
      ],
      out_specs=pl.BlockSpec((_BLK,), lambda i: (i,)),
      out_shape=jax.ShapeDtypeStruct((TC_COLS,), jnp.float32),
  )(tableT, W.reshape(LATENT, 1))


def _make_sc_sweep():
  @functools.partial(
      pl.kernel,
      mesh=_mesh(),
      compiler_params=pltpu.CompilerParams(use_tc_tiling_on_sc=True),
      out_type=jax.ShapeDtypeStruct((SC_COLS,), jnp.float32),
      scratch_types=[
          pltpu.VMEM((LATENT,), jnp.float32),
          pltpu.VMEM((LATENT, SLAB), jnp.float32),
          pltpu.VMEM((NSLAB * SLAB,), jnp.float32),
          pltpu.SemaphoreType.DMA,
      ],
  )
  def sweep_k(w_hbm, tableT_hbm, out_hbm, w_v, slab_v, acc_v, sem):
    wid = lax.axis_index("s") * NC + lax.axis_index("c")
    pltpu.sync_copy(w_hbm, w_v)
    cbase = wid * NSLAB * SLAB

    def slab_body(n, _):
      pltpu.async_copy(
          tableT_hbm.at[:, pl.ds(cbase + n * SLAB, SLAB)], slab_v,
          sem).wait()
      for cg in range(SLAB // 16):
        acc = jnp.zeros((16,), jnp.float32)
        for q in range(LATENT // 16):
          w16 = w_v[pl.ds(q * 16, 16)]
          for r2 in range(16):
            wb = jnp.take(w16, jnp.full((16,), r2, jnp.int32))
            acc = acc + slab_v[q * 16 + r2, pl.ds(cg * 16, 16)] * wb
        acc_v[pl.ds(n * SLAB + cg * 16, 16)] = acc
      return _

    lax.fori_loop(0, NSLAB, slab_body, None)
    pltpu.sync_copy(acc_v, out_hbm.at[pl.ds(cbase, NSLAB * SLAB)])

  return sweep_k


_sc_sweep = _make_sc_sweep()


def _make_select():
  @functools.partial(
      pl.kernel,
      mesh=_mesh(),
      out_type=jax.ShapeDtypeStruct((BATCH,), jnp.float32),
      scratch_types=[
          pltpu.VMEM((B_PER_W,), jnp.int32),
          pltpu.VMEM((B_PER_W,), jnp.int32),
          pltpu.VMEM((B_PER_W,), jnp.int32),
          pltpu.VMEM((B_PER_W,), jnp.float32),
          pltpu.VMEM((B_PER_W,), jnp.float32),
          pltpu.VMEM((16,), jnp.float32),
          pltpu.SemaphoreType.DMA,
      ],
  )
  def select_k(idx_hbm, ssc_hbm, stc_hbm, b_hbm, out_hbm, idx_v, ia_v,
               ib_v, ga_v, gb_v, b_v, sem):
    wid = lax.axis_index("s") * NC + lax.axis_index("c")
    base = wid * B_PER_W
    pltpu.sync_copy(idx_hbm.at[pl.ds(base, B_PER_W)], idx_v)
    pltpu.sync_copy(b_hbm, b_v)

    def split(g, _):
      v = idx_v[pl.ds(g * 16, 16)]
      ia_v[pl.ds(g * 16, 16)] = jnp.minimum(v, SC_COLS - 1)
      ib_v[pl.ds(g * 16, 16)] = jnp.clip(v - SC_COLS, 0, TC_COLS - 1)
      return _

    lax.fori_loop(0, B_PER_W // 16, split, None)
    pltpu.async_copy(ssc_hbm.at[ia_v], ga_v, sem).wait()
    pltpu.async_copy(stc_hbm.at[ib_v], gb_v, sem).wait()

    def fin(g, _):
      v = idx_v[pl.ds(g * 16, 16)]
      a = ga_v[pl.ds(g * 16, 16)]
      b2 = gb_v[pl.ds(g * 16, 16)]
      s = jnp.where(v < SC_COLS, a, b2) + b_v[pl.ds(0, 16)]
      ga_v[pl.ds(g * 16, 16)] = 1.0 / (1.0 + jnp.exp(-s))
      return _

    lax.fori_loop(0, B_PER_W // 16, fin, None)
    pltpu.sync_copy(ga_v, out_hbm.at[pl.ds(base, B_PER_W)])

  return select_k


_select = _make_select()


@jax.jit
def kernel(item_indices, emb_table, W, b):
  idx0 = (item_indices - 1).astype(jnp.int32)
  tableT = emb_table.T
  w_flat = W.reshape(LATENT)
  s_sc = _sc_sweep(w_flat, tableT)
  s_tc = _tc_sweep(tableT, W)
  b16 = jnp.broadcast_to(b, (16,))
  return _select(idx0, s_sc, s_tc, b16)


# speedup vs baseline: 1.8047x; 1.4947x over previous
import functools

import jax
import jax.numpy as jnp
from jax import lax
from jax.experimental import pallas as pl
from jax.experimental.pallas import tpu as pltpu
from jax.experimental.pallas import tpu_sc as plsc

NUM_ITEMS = 1000000
LATENT = 64
BATCH = 16384

NC = 2
NS = 16
NW = NC * NS
B_PER_W = BATCH // NW

SC_COLS = 262144
SLAB = 128
NSLAB = SC_COLS // (SLAB * NW)
TC_COLS = NUM_ITEMS - SC_COLS

_BLK = 32768


def _mesh():
  return plsc.VectorSubcoreMesh(
      core_axis_name="c", subcore_axis_name="s", num_cores=NC,
      num_subcores=NS)


def _mv_body(xT_ref, w_ref, o_ref):
  x = xT_ref[...]
  w = w_ref[...]
  o_ref[...] = jnp.sum(x * w, axis=0)


def _tc_sweep(tableT, W):
  grid = (pl.cdiv(TC_COLS, _BLK),)
  return pl.pallas_call(
      _mv_body,
      grid=grid,
      in_specs=[
          pl.BlockSpec((LATENT, _BLK), lambda i: (0, i + SC_COLS // _BLK)),
          pl.BlockSpec((LATENT, 1), lambda i: (0, 0)),
      ],
      out_specs=pl.BlockSpec((_BLK,), lambda i: (i,)),
      out_shape=jax.ShapeDtypeStruct((TC_COLS,), jnp.float32),
  )(tableT, W.reshape(LATENT, 1))


def _make_sc_sweep():
  @functools.partial(
      pl.kernel,
      mesh=_mesh(),
      compiler_params=pltpu.CompilerParams(use_tc_tiling_on_sc=True),
      out_type=jax.ShapeDtypeStruct((SC_COLS,), jnp.float32),
      scratch_types=[
          pltpu.VMEM((LATENT,), jnp.float32),
          pltpu.VMEM((LATENT, SLAB), jnp.float32),
          pltpu.VMEM((LATENT, SLAB), jnp.float32),
          pltpu.VMEM((NSLAB * SLAB,), jnp.float32),
          pltpu.SemaphoreType.DMA,
          pltpu.SemaphoreType.DMA,
      ],
  )
  def sweep_k(w_hbm, tableT_hbm, out_hbm, w_v, slab0_v, slab1_v, acc_v,
              sem0, sem1):
    wid = lax.axis_index("s") * NC + lax.axis_index("c")
    pltpu.sync_copy(w_hbm, w_v)
    cbase = wid * NSLAB * SLAB
    bufs = (slab0_v, slab1_v)
    sems = (sem0, sem1)

    def issue(m, ph):
      pltpu.async_copy(
          tableT_hbm.at[:, pl.ds(cbase + m * SLAB, SLAB)], bufs[ph],
          sems[ph])

    def compute(buf, m):
      for cg in range(SLAB // 16):
        acc = jnp.zeros((16,), jnp.float32)
        for q in range(LATENT // 16):
          w16 = w_v[pl.ds(q * 16, 16)]
          for r2 in range(16):
            wb = jnp.take(w16, jnp.full((16,), r2, jnp.int32))
            acc = acc + buf[q * 16 + r2, pl.ds(cg * 16, 16)] * wb
        acc_v[pl.ds(m * SLAB + cg * 16, 16)] = acc

    issue(0, 0)
    issue(1, 1)

    def pair_body(p, _):
      for ph in range(2):
        m = p * 2 + ph
        pltpu.make_async_copy(
            tableT_hbm.at[:, pl.ds(0, SLAB)], bufs[ph], sems[ph]).wait()
        compute(bufs[ph], m)

        @pl.when(m + 2 < NSLAB)
        def _issue_next():
          issue(m + 2, ph)
      return _

    lax.fori_loop(0, NSLAB // 2, pair_body, None)
    pltpu.sync_copy(acc_v, out_hbm.at[pl.ds(cbase, NSLAB * SLAB)])

  return sweep_k


_sc_sweep = _make_sc_sweep()


def _make_select():
  @functools.partial(
      pl.kernel,
      mesh=_mesh(),
      out_type=jax.ShapeDtypeStruct((BATCH,), jnp.float32),
      scratch_types=[
          pltpu.VMEM((B_PER_W,), jnp.int32),
          pltpu.VMEM((B_PER_W,), jnp.int32),
          pltpu.VMEM((B_PER_W,), jnp.int32),
          pltpu.VMEM((B_PER_W,), jnp.float32),
          pltpu.VMEM((B_PER_W,), jnp.float32),
          pltpu.VMEM((16,), jnp.float32),
          pltpu.SemaphoreType.DMA,
      ],
  )
  def select_k(idx_hbm, ssc_hbm, stc_hbm, b_hbm, out_hbm, idx_v, ia_v,
               ib_v, ga_v, gb_v, b_v, sem):
    wid = lax.axis_index("s") * NC + lax.axis_index("c")
    base = wid * B_PER_W
    pltpu.sync_copy(idx_hbm.at[pl.ds(base, B_PER_W)], idx_v)
    pltpu.sync_copy(b_hbm, b_v)

    def split(g, _):
      v = idx_v[pl.ds(g * 16, 16)]
      dummy = base + g * 16 + lax.iota(jnp.int32, 16)
      ia_v[pl.ds(g * 16, 16)] = jnp.where(v < SC_COLS, v, dummy)
      ib_v[pl.ds(g * 16, 16)] = jnp.where(v >= SC_COLS, v - SC_COLS,
                                          dummy)
      return _

    lax.fori_loop(0, B_PER_W // 16, split, None)
    pltpu.async_copy(ssc_hbm.at[ia_v], ga_v, sem).wait()
    pltpu.async_copy(stc_hbm.at[ib_v], gb_v, sem).wait()

    def fin(g, _):
      v = idx_v[pl.ds(g * 16, 16)]
      a = ga_v[pl.ds(g * 16, 16)]
      b2 = gb_v[pl.ds(g * 16, 16)]
      s = jnp.where(v < SC_COLS, a, b2) + b_v[pl.ds(0, 16)]
      ga_v[pl.ds(g * 16, 16)] = 1.0 / (1.0 + jnp.exp(-s))
      return _

    lax.fori_loop(0, B_PER_W // 16, fin, None)
    pltpu.sync_copy(ga_v, out_hbm.at[pl.ds(base, B_PER_W)])

  return select_k


_select = _make_select()


@jax.jit
def kernel(item_indices, emb_table, W, b):
  idx0 = (item_indices - 1).astype(jnp.int32)
  tableT = emb_table.T
  w_flat = W.reshape(LATENT)
  s_sc = _sc_sweep(w_flat, tableT)
  s_tc = _tc_sweep(tableT, W)
  b16 = jnp.broadcast_to(b, (16,))
  return _select(idx0, s_sc, s_tc, b16)


# speedup vs baseline: 2.0936x; 1.1601x over previous
import functools

import jax
import jax.numpy as jnp
from jax import lax
from jax.experimental import pallas as pl
from jax.experimental.pallas import tpu as pltpu
from jax.experimental.pallas import tpu_sc as plsc

NUM_ITEMS = 1000000
LATENT = 64
BATCH = 16384

NC = 2
NS = 16
NW = NC * NS
B_PER_W = BATCH // NW

SC_COLS = 262144
SLAB = 128
NSLAB = SC_COLS // (SLAB * NW)
TC_COLS = NUM_ITEMS - SC_COLS

_BLK = 32768


def _mesh():
  return plsc.VectorSubcoreMesh(
      core_axis_name="c", subcore_axis_name="s", num_cores=NC,
      num_subcores=NS)


def _mv_body(xT_ref, w_ref, o_ref):
  x = xT_ref[...]
  w = w_ref[...]
  o_ref[...] = jnp.sum(x * w, axis=0)


def _tc_sweep(tableT, W):
  grid = (pl.cdiv(TC_COLS, _BLK),)
  return pl.pallas_call(
      _mv_body,
      grid=grid,
      in_specs=[
          pl.BlockSpec((LATENT, _BLK), lambda i: (0, i + SC_COLS // _BLK)),
          pl.BlockSpec((LATENT, 1), lambda i: (0, 0)),
      ],
      out_specs=pl.BlockSpec((_BLK,), lambda i: (i,)),
      out_shape=jax.ShapeDtypeStruct((TC_COLS,), jnp.float32),
  )(tableT, W.reshape(LATENT, 1))


def _make_sc_sweep():
  @functools.partial(
      pl.kernel,
      mesh=_mesh(),
      compiler_params=pltpu.CompilerParams(use_tc_tiling_on_sc=True),
      out_type=jax.ShapeDtypeStruct((SC_COLS,), jnp.float32),
      scratch_types=[
          pltpu.VMEM((LATENT,), jnp.float32),
          pltpu.VMEM((LATENT, SLAB), jnp.float32),
          pltpu.VMEM((LATENT, SLAB), jnp.float32),
          pltpu.VMEM((NSLAB * SLAB,), jnp.float32),
          pltpu.SemaphoreType.DMA,
          pltpu.SemaphoreType.DMA,
      ],
  )
  def sweep_k(w_hbm, tableT_hbm, out_hbm, w_v, slab0_v, slab1_v, acc_v,
              sem0, sem1):
    wid = lax.axis_index("s") * NC + lax.axis_index("c")
    pltpu.sync_copy(w_hbm, w_v)
    cbase = wid * NSLAB * SLAB
    bufs = (slab0_v, slab1_v)
    sems = (sem0, sem1)

    def issue(m, ph):
      pltpu.async_copy(
          tableT_hbm.at[:, pl.ds(cbase + m * SLAB, SLAB)], bufs[ph],
          sems[ph])

    def compute(buf, m):
      accs = [jnp.zeros((16,), jnp.float32) for _ in range(SLAB // 16)]
      for q in range(LATENT // 16):
        w16 = w_v[pl.ds(q * 16, 16)]
        for r2 in range(16):
          wb = jnp.take(w16, jnp.full((16,), r2, jnp.int32))
          for cg in range(SLAB // 16):
            accs[cg] = accs[cg] + buf[q * 16 + r2,
                                      pl.ds(cg * 16, 16)] * wb
      for cg in range(SLAB // 16):
        acc_v[pl.ds(m * SLAB + cg * 16, 16)] = accs[cg]

    issue(0, 0)
    issue(1, 1)

    def pair_body(p, _):
      for ph in range(2):
        m = p * 2 + ph
        pltpu.make_async_copy(
            tableT_hbm.at[:, pl.ds(0, SLAB)], bufs[ph], sems[ph]).wait()
        compute(bufs[ph], m)

        @pl.when(m + 2 < NSLAB)
        def _issue_next():
          issue(m + 2, ph)
      return _

    lax.fori_loop(0, NSLAB // 2, pair_body, None)
    pltpu.sync_copy(acc_v, out_hbm.at[pl.ds(cbase, NSLAB * SLAB)])

  return sweep_k


_sc_sweep = _make_sc_sweep()


def _make_select():
  @functools.partial(
      pl.kernel,
      mesh=_mesh(),
      out_type=jax.ShapeDtypeStruct((BATCH,), jnp.float32),
      scratch_types=[
          pltpu.VMEM((B_PER_W,), jnp.int32),
          pltpu.VMEM((B_PER_W,), jnp.int32),
          pltpu.VMEM((B_PER_W,), jnp.int32),
          pltpu.VMEM((B_PER_W,), jnp.float32),
          pltpu.VMEM((B_PER_W,), jnp.float32),
          pltpu.VMEM((16,), jnp.float32),
          pltpu.SemaphoreType.DMA,
      ],
  )
  def select_k(idx_hbm, ssc_hbm, stc_hbm, b_hbm, out_hbm, idx_v, ia_v,
               ib_v, ga_v, gb_v, b_v, sem):
    wid = lax.axis_index("s") * NC + lax.axis_index("c")
    base = wid * B_PER_W
    pltpu.sync_copy(idx_hbm.at[pl.ds(base, B_PER_W)], idx_v)
    pltpu.sync_copy(b_hbm, b_v)

    def split(g, _):
      v = idx_v[pl.ds(g * 16, 16)]
      dummy = base + g * 16 + lax.iota(jnp.int32, 16)
      ia_v[pl.ds(g * 16, 16)] = jnp.where(v < SC_COLS, v, dummy)
      ib_v[pl.ds(g * 16, 16)] = jnp.where(v >= SC_COLS, v - SC_COLS,
                                          dummy)
      return _

    lax.fori_loop(0, B_PER_W // 16, split, None)
    pltpu.async_copy(ssc_hbm.at[ia_v], ga_v, sem).wait()
    pltpu.async_copy(stc_hbm.at[ib_v], gb_v, sem).wait()

    def fin(g, _):
      v = idx_v[pl.ds(g * 16, 16)]
      a = ga_v[pl.ds(g * 16, 16)]
      b2 = gb_v[pl.ds(g * 16, 16)]
      s = jnp.where(v < SC_COLS, a, b2) + b_v[pl.ds(0, 16)]
      ga_v[pl.ds(g * 16, 16)] = 1.0 / (1.0 + jnp.exp(-s))
      return _

    lax.fori_loop(0, B_PER_W // 16, fin, None)
    pltpu.sync_copy(ga_v, out_hbm.at[pl.ds(base, B_PER_W)])

  return select_k


_select = _make_select()


@jax.jit
def kernel(item_indices, emb_table, W, b):
  idx0 = (item_indices - 1).astype(jnp.int32)
  tableT = emb_table.T
  w_flat = W.reshape(LATENT)
  s_sc = _sc_sweep(w_flat, tableT)
  s_tc = _tc_sweep(tableT, W)
  b16 = jnp.broadcast_to(b, (16,))
  return _select(idx0, s_sc, s_tc, b16)


# speedup vs baseline: 2.5843x; 1.2344x over previous
import functools

import jax
import jax.numpy as jnp
from jax import lax
from jax.experimental import pallas as pl
from jax.experimental.pallas import tpu as pltpu
from jax.experimental.pallas import tpu_sc as plsc

NUM_ITEMS = 1000000
LATENT = 64
BATCH = 16384

NC = 2
NS = 16
NW = NC * NS
B_PER_W = BATCH // NW

_BLK = 32768


def _mv_body(xT_ref, w_ref, b_ref, o_ref):
  x = xT_ref[...]
  w = w_ref[...]
  o_ref[...] = jnp.sum(x * w, axis=0) + b_ref[0]


def _matvec_stage(tableT, W, b):
  grid = (pl.cdiv(NUM_ITEMS, _BLK),)
  return pl.pallas_call(
      _mv_body,
      grid=grid,
      in_specs=[
          pl.BlockSpec((LATENT, _BLK), lambda i: (0, i)),
          pl.BlockSpec((LATENT, 1), lambda i: (0, 0)),
          pl.BlockSpec(memory_space=pltpu.SMEM),
      ],
      out_specs=pl.BlockSpec((_BLK,), lambda i: (i,)),
      out_shape=jax.ShapeDtypeStruct((NUM_ITEMS,), jnp.float32),
  )(tableT, W.reshape(LATENT, 1), b)


def _make_select():
  mesh = plsc.VectorSubcoreMesh(
      core_axis_name="c", subcore_axis_name="s", num_cores=NC,
      num_subcores=NS)

  @functools.partial(
      pl.kernel,
      mesh=mesh,
      out_type=jax.ShapeDtypeStruct((BATCH,), jnp.float32),
      scratch_types=[
          pltpu.VMEM((B_PER_W,), jnp.int32),
          pltpu.VMEM((B_PER_W,), jnp.float32),
          pltpu.SemaphoreType.DMA,
      ],
  )
  def select_k(idx_hbm, s_hbm, out_hbm, idx_v, g_v, sem):
    cid = lax.axis_index("c")
    sid = lax.axis_index("s")
    wid = sid * NC + cid
    base = wid * B_PER_W
    pltpu.sync_copy(idx_hbm.at[pl.ds(base, B_PER_W)], idx_v)
    pltpu.async_copy(s_hbm.at[idx_v], g_v, sem).wait()
    for g in range(B_PER_W // 16):
      v = g_v[pl.ds(g * 16, 16)]
      r = 1.0 / (1.0 + jnp.exp(-v))
      g_v[pl.ds(g * 16, 16)] = r
    pltpu.sync_copy(g_v, out_hbm.at[pl.ds(base, B_PER_W)])

  return select_k


_select = _make_select()


@jax.jit
def kernel(item_indices, emb_table, W, b):
  idx0 = (item_indices - 1).astype(jnp.int32)
  s = _matvec_stage(emb_table.T, W, b)
  return _select(idx0, s)


# speedup vs baseline: 2.6107x; 1.0102x over previous
import functools

import jax
import jax.numpy as jnp
from jax import lax
from jax.experimental import pallas as pl
from jax.experimental.pallas import tpu as pltpu
from jax.experimental.pallas import tpu_sc as plsc

NUM_ITEMS = 1000000
LATENT = 64
BATCH = 16384

NC = 2
NS = 16
NW = NC * NS
B_PER_W = BATCH // NW

_BLK = 40960


def _mv_body(xT_ref, w_ref, b_ref, o_ref):
  x = xT_ref[...]
  w = w_ref[...]
  o_ref[...] = jnp.sum(x * w, axis=0) + b_ref[0]


def _matvec_stage(tableT, W, b):
  grid = (pl.cdiv(NUM_ITEMS, _BLK),)
  return pl.pallas_call(
      _mv_body,
      grid=grid,
      in_specs=[
          pl.BlockSpec((LATENT, _BLK), lambda i: (0, i)),
          pl.BlockSpec((LATENT, 1), lambda i: (0, 0)),
          pl.BlockSpec(memory_space=pltpu.SMEM),
      ],
      out_specs=pl.BlockSpec((_BLK,), lambda i: (i,)),
      out_shape=jax.ShapeDtypeStruct((NUM_ITEMS,), jnp.float32),
  )(tableT, W.reshape(LATENT, 1), b)


def _make_select():
  mesh = plsc.VectorSubcoreMesh(
      core_axis_name="c", subcore_axis_name="s", num_cores=NC,
      num_subcores=NS)

  @functools.partial(
      pl.kernel,
      mesh=mesh,
      out_type=jax.ShapeDtypeStruct((BATCH,), jnp.float32),
      scratch_types=[
          pltpu.VMEM((B_PER_W,), jnp.int32),
          pltpu.VMEM((B_PER_W,), jnp.float32),
          pltpu.SemaphoreType.DMA,
      ],
  )
  def select_k(idx_hbm, s_hbm, out_hbm, idx_v, g_v, sem):
    cid = lax.axis_index("c")
    sid = lax.axis_index("s")
    wid = sid * NC + cid
    base = wid * B_PER_W
    pltpu.sync_copy(idx_hbm.at[pl.ds(base, B_PER_W)], idx_v)
    pltpu.async_copy(s_hbm.at[idx_v], g_v, sem).wait()
    for g in range(B_PER_W // 16):
      v = g_v[pl.ds(g * 16, 16)]
      r = 1.0 / (1.0 + jnp.exp(-v))
      g_v[pl.ds(g * 16, 16)] = r
    pltpu.sync_copy(g_v, out_hbm.at[pl.ds(base, B_PER_W)])

  return select_k


_select = _make_select()


@jax.jit
def kernel(item_indices, emb_table, W, b):
  idx0 = (item_indices - 1).astype(jnp.int32)
  s = _matvec_stage(emb_table.T, W, b)
  return _select(idx0, s)
